# Initial kernel scaffold; baseline (speedup 1.0000x reference)
import jax
import jax.numpy as jnp
from jax.experimental import pallas as pl


def kernel(x, emb_tables, lin_tables, bias):
    raise NotImplementedError("write your pallas kernel here")


# speedup vs baseline: 1.4280x; 1.4280x over previous
import jax
import jax.numpy as jnp
from jax import lax
from jax.experimental import pallas as pl
from jax.experimental.pallas import tpu as pltpu
from jax.experimental.pallas import tpu_sc as plsc

_F = 26
_V = 100000
_D = 32
_B = 16384

_L = 16
_NW = 32
_CB = _B // _NW
_GS = _L
_GROUPS = _CB // _GS
_CHUNK = 104
_CPG = _GS * _F // _CHUNK
_RG = _GS * _F
_CPW = _CB * _F // _CHUNK


def _make_fm_kernel():
    scmesh = plsc.VectorSubcoreMesh(core_axis_name="c", subcore_axis_name="s")

    def body(dma_hbm, raw_hbm, emb_hbm, lin_hbm, bias_hbm, out_hbm,
             dma_v, raw_v, rows_v, lin_v, out_v, bias_v,
             sem_e0, sem_e1, sem_l0, sem_l1):
        c = lax.axis_index("c")
        s = lax.axis_index("s")
        wid = s * 2 + c
        pltpu.sync_copy(dma_hbm.at[pl.ds(wid * _CPW, _CPW), :], dma_v)
        pltpu.sync_copy(bias_hbm, bias_v)
        sems_e = (sem_e0, sem_e1)
        sems_l = (sem_l0, sem_l1)

        def fire(g, par):
            pltpu.sync_copy(raw_hbm.at[pl.ds(wid * _CPW + g * _CPG, _CPG), :],
                            raw_v.at[par])
            for j in range(_CPG):
                ch = g * _CPG + j
                pltpu.async_copy(emb_hbm.at[dma_v.at[ch]],
                                 rows_v.at[par].at[pl.ds(j * _CHUNK, _CHUNK), :],
                                 sems_e[par])
                pltpu.async_copy(lin_hbm.at[raw_v.at[par].at[j]],
                                 lin_v.at[par].at[j],
                                 sems_l[par])

        def drain(par):
            for j in range(_CPG):
                pltpu.make_async_copy(emb_hbm.at[pl.ds(0, _CHUNK), :],
                                      rows_v.at[par].at[pl.ds(j * _CHUNK, _CHUNK), :],
                                      sems_e[par]).wait()
                pltpu.make_async_copy(lin_hbm.at[pl.ds(0, _CHUNK)],
                                      lin_v.at[par].at[j],
                                      sems_l[par]).wait()

        iota = lax.iota(jnp.int32, _L)
        iota26 = iota * _F
        zero = jnp.zeros((_L,), jnp.float32)

        def compute(g, par):
            rows2 = rows_v.at[par]
            raw2 = raw_v.at[par]
            lin2 = lin_v.at[par]

            def fbody(f, carry):
                accs = carry[:_D]
                acc_sq = carry[_D]
                lin_acc = carry[_D + 1]
                r = iota26 + f
                rc = r // _CHUNK
                rw = r % _CHUNK
                ivraw = plsc.load_gather(raw2, [rc, rw])
                colb = (ivraw & 3) * _D
                lin_acc = lin_acc + plsc.load_gather(lin2, [rc, rw])
                new_accs = []
                for d in range(_D):
                    v = plsc.load_gather(rows2, [r, colb + d])
                    new_accs.append(accs[d] + v)
                    acc_sq = acc_sq + v * v
                return (*new_accs, acc_sq, lin_acc)

            init = (zero,) * (_D + 2)
            res = lax.fori_loop(0, _F, fbody, init)
            accs, acc_sq, lin_acc = res[:_D], res[_D], res[_D + 1]
            ss = accs[0] * accs[0]
            for d in range(1, _D):
                ss = ss + accs[d] * accs[d]
            logit = bias_v[...] + lin_acc + 0.5 * (ss - acc_sq)
            out_v[pl.ds(g * _GS, _L)] = 1.0 / (1.0 + jnp.exp(-logit))

        fire(0, 0)

        def gbody(g2, carry):
            fire(2 * g2 + 1, 1)
            drain(0)
            compute(2 * g2, 0)

            @pl.when(g2 < _GROUPS // 2 - 1)
            def _():
                fire(2 * g2 + 2, 0)

            drain(1)
            compute(2 * g2 + 1, 1)
            return carry

        lax.fori_loop(0, _GROUPS // 2, gbody, 0)
        pltpu.sync_copy(out_v, out_hbm.at[pl.ds(wid * _CB, _CB)])

    return pl.kernel(
        body,
        out_type=jax.ShapeDtypeStruct((_B,), jnp.float32),
        mesh=scmesh,
        scratch_types=[
            pltpu.VMEM((_CPW, _CHUNK), jnp.int32),
            pltpu.VMEM((2, _CPG, _CHUNK), jnp.int32),
            pltpu.VMEM((2, _RG, 4 * _D), jnp.float32),
            pltpu.VMEM((2, _CPG, _CHUNK), jnp.float32),
            pltpu.VMEM((_CB,), jnp.float32),
            pltpu.VMEM((_L,), jnp.float32),
            pltpu.SemaphoreType.DMA,
            pltpu.SemaphoreType.DMA,
            pltpu.SemaphoreType.DMA,
            pltpu.SemaphoreType.DMA,
        ],
        compiler_params=pltpu.CompilerParams(needs_layout_passes=False),
    )


def kernel(x, emb_tables, lin_tables, bias):
    F, V, D = emb_tables.shape
    idx = (x + jnp.arange(F, dtype=jnp.int32)[None, :] * V).reshape(-1)
    raw2d = idx.reshape(-1, _CHUNK)
    dma2d = (idx >> 2).reshape(-1, _CHUNK)
    emb_wide = emb_tables.reshape(F * V // 4, 4 * D)
    lin_flat = lin_tables.reshape(F * V)
    bias16 = jnp.broadcast_to(bias.astype(jnp.float32), (_L,))
    out = _make_fm_kernel()(dma2d, raw2d, emb_wide, lin_flat, bias16)
    return out.reshape(-1, 1)
